# baseline (speedup 1.0000x reference)
import functools

import jax
import jax.numpy as jnp
from jax import lax
from jax.experimental import pallas as pl
from jax.experimental.pallas import tpu as pltpu
from jax.experimental.pallas import tpu_sc as plsc

N_NODES = 10000
N_EDGES = 320000
D_FEAT = 128
HIDDEN = 128
D_HALF = D_FEAT // 2

NC = 2
NS = 16
NW = NC * NS

EDGES_PER_TILE = N_EDGES // NS
CHUNK = 80
NCHUNK = EDGES_PER_TILE // CHUNK
N_PAD = 10240
ROWS_PER_TILE = N_PAD // NS
NBUF = 5
NGROUP = NCHUNK // NBUF
LANES = 16

_sc_mesh = plsc.VectorSubcoreMesh(core_axis_name="c", subcore_axis_name="s")


@functools.partial(
    pl.kernel,
    out_type=jax.ShapeDtypeStruct((N_PAD, D_FEAT), jnp.float32),
    mesh=_sc_mesh,
    compiler_params=pltpu.CompilerParams(use_tc_tiling_on_sc=False),
    scratch_types=[
        pltpu.VMEM_SHARED((N_PAD, D_HALF), jnp.float32),
        pltpu.VMEM((EDGES_PER_TILE,), jnp.int32),
        pltpu.VMEM((EDGES_PER_TILE,), jnp.int32),
        pltpu.VMEM((NCHUNK, CHUNK), jnp.int32),
    ]
    + [pltpu.VMEM((CHUNK, D_HALF), jnp.float32) for _ in range(NBUF)]
    + [pltpu.SemaphoreType.DMA for _ in range(2 * NBUF)],
)
def _sc_segment_sum(x2_hbm, edges_hbm, zeros_hbm, out_hbm,
                    acc, src_v, dst_v1, dst_v, *bufs_and_sems):
    bufs = bufs_and_sems[:NBUF]
    gsems = bufs_and_sems[NBUF:2 * NBUF]
    ssems = bufs_and_sems[2 * NBUF:]
    cid = lax.axis_index("c")
    sid = lax.axis_index("s")

    pltpu.sync_copy(zeros_hbm, acc.at[pl.ds(sid * ROWS_PER_TILE, ROWS_PER_TILE)])
    pltpu.sync_copy(edges_hbm.at[pl.ds(sid * EDGES_PER_TILE, EDGES_PER_TILE)],
                    src_v)
    pltpu.sync_copy(
        edges_hbm.at[pl.ds(N_EDGES + sid * EDGES_PER_TILE, EDGES_PER_TILE)],
        dst_v1)

    PER_ROW = CHUNK // LANES

    XU = 5

    def xform(k0, carry):
        for u in range(XU):
            k = k0 * XU + u
            v = src_v[pl.ds(k * LANES, LANES)]
            src_v[pl.ds(k * LANES, LANES)] = 2 * v + cid
            d = dst_v1[pl.ds(k * LANES, LANES)]
            dst_v[k // PER_ROW, pl.ds((k % PER_ROW) * LANES, LANES)] = d
        return carry

    lax.fori_loop(0, EDGES_PER_TILE // LANES // XU, xform, 0)
    plsc.subcore_barrier()

    for b in range(NBUF - 1):
        pltpu.async_copy(
            x2_hbm.at[src_v.at[pl.ds(b * CHUNK, CHUNK)]], bufs[b], gsems[b])

    def group(g, carry):
        for u in range(NBUF):
            j = NBUF * g + u
            pltpu.make_async_copy(
                x2_hbm.at[src_v.at[pl.ds(j * CHUNK, CHUNK)]], bufs[u],
                gsems[u]).wait()
            pltpu.async_copy(bufs[u], acc.at[dst_v.at[j]], ssems[u], add=True)
            un = (u + NBUF - 1) % NBUF

            @pl.when(j + NBUF - 1 < NCHUNK)
            def _():
                def _wait_prev():
                    pltpu.make_async_copy(
                        bufs[un], acc.at[dst_v.at[j - 1]], ssems[un]).wait()

                if u == 0:
                    pl.when(g > 0)(_wait_prev)
                else:
                    _wait_prev()
                pltpu.async_copy(
                    x2_hbm.at[src_v.at[pl.ds((j + NBUF - 1) * CHUNK, CHUNK)]],
                    bufs[un], gsems[un])
        return carry

    lax.fori_loop(0, NGROUP, group, 0)
    for u in range(NBUF):
        pltpu.make_async_copy(
            bufs[u], acc.at[dst_v.at[NCHUNK - NBUF + u]], ssems[u]).wait()
    plsc.subcore_barrier()

    pltpu.sync_copy(
        acc.at[pl.ds(sid * ROWS_PER_TILE, ROWS_PER_TILE)],
        out_hbm.at[pl.ds(sid * ROWS_PER_TILE, ROWS_PER_TILE),
                   pl.ds(cid * D_HALF, D_HALF)],
    )


BLK = 2000


def _mlp_body(eps_ref, x_ref, agg_ref, w1_ref, b1_ref, w2_ref, b2_ref,
              o_ref):
    coeff = 2.0 + eps_ref[0, 0]
    h = coeff * x_ref[...] + agg_ref[...]
    a = jnp.maximum(
        jnp.dot(h, w1_ref[...], preferred_element_type=jnp.float32)
        + b1_ref[...], 0.0)
    o_ref[...] = (
        jnp.dot(a, w2_ref[...], preferred_element_type=jnp.float32)
        + b2_ref[...])


def _mlp(eps, x, agg, W1, b1, W2, b2):
    grid = (N_NODES // BLK,)
    return pl.pallas_call(
        _mlp_body,
        grid=grid,
        in_specs=[
            pl.BlockSpec((1, 1), lambda i: (0, 0)),
            pl.BlockSpec((BLK, D_FEAT), lambda i: (i, 0)),
            pl.BlockSpec((BLK, D_FEAT), lambda i: (i, 0)),
            pl.BlockSpec((D_FEAT, HIDDEN), lambda i: (0, 0)),
            pl.BlockSpec((1, HIDDEN), lambda i: (0, 0)),
            pl.BlockSpec((HIDDEN, HIDDEN), lambda i: (0, 0)),
            pl.BlockSpec((1, HIDDEN), lambda i: (0, 0)),
        ],
        out_specs=pl.BlockSpec((BLK, HIDDEN), lambda i: (i, 0)),
        out_shape=jax.ShapeDtypeStruct((N_NODES, HIDDEN), jnp.float32),
    )(eps, x, agg, W1, b1, W2, b2)


def kernel(x, edge_index, edge_attr, eps, W1, b1, W2, b2):
    edges = edge_index.astype(jnp.int32).reshape(2 * N_EDGES)
    x2 = x.reshape(2 * N_NODES, D_HALF)
    zeros = jnp.zeros((ROWS_PER_TILE, D_HALF), dtype=jnp.float32)

    agg = _sc_segment_sum(x2, edges, zeros)

    eps2 = eps.reshape(1, 1)
    b1r = b1.reshape(1, HIDDEN)
    b2r = b2.reshape(1, HIDDEN)
    return _mlp(eps2, x, agg, W1, b1r, W2, b2r)


# speedup vs baseline: 1.0372x; 1.0372x over previous
import functools

import jax
import jax.numpy as jnp
from jax import lax
from jax.experimental import pallas as pl
from jax.experimental.pallas import tpu as pltpu
from jax.experimental.pallas import tpu_sc as plsc

N_NODES = 10000
N_EDGES = 320000
D_FEAT = 128
HIDDEN = 128
D_HALF = D_FEAT // 2

NC = 2
NS = 16
NW = NC * NS

EDGES_PER_TILE = N_EDGES // NS
CHUNK = 80
NCHUNK = EDGES_PER_TILE // CHUNK
N_PAD = 10240
ROWS_PER_TILE = N_PAD // NS
NBUF = 5
NGROUP = NCHUNK // NBUF
LANES = 16

_sc_mesh = plsc.VectorSubcoreMesh(core_axis_name="c", subcore_axis_name="s")


@functools.partial(
    pl.kernel,
    out_type=jax.ShapeDtypeStruct((N_PAD, D_FEAT), jnp.float32),
    mesh=_sc_mesh,
    compiler_params=pltpu.CompilerParams(use_tc_tiling_on_sc=False),
    scratch_types=[
        pltpu.VMEM_SHARED((N_PAD, D_HALF), jnp.float32),
        pltpu.VMEM((EDGES_PER_TILE,), jnp.int32),
        pltpu.VMEM((EDGES_PER_TILE,), jnp.int32),
        pltpu.VMEM((NCHUNK, CHUNK), jnp.int32),
    ]
    + [pltpu.VMEM((CHUNK, D_HALF), jnp.float32) for _ in range(NBUF)]
    + [pltpu.SemaphoreType.DMA for _ in range(NBUF)],
)
def _sc_segment_sum(x2_hbm, edges_hbm, zeros_hbm, out_hbm,
                    acc, src_v, dst_v1, dst_v, *bufs_and_sems):
    bufs = bufs_and_sems[:NBUF]
    sems = bufs_and_sems[NBUF:]
    cid = lax.axis_index("c")
    sid = lax.axis_index("s")

    pltpu.async_copy(zeros_hbm,
                     acc.at[pl.ds(sid * ROWS_PER_TILE, ROWS_PER_TILE)],
                     sems[0])
    pltpu.async_copy(
        edges_hbm.at[pl.ds(sid * EDGES_PER_TILE, EDGES_PER_TILE)], src_v,
        sems[1])
    pltpu.async_copy(
        edges_hbm.at[pl.ds(N_EDGES + sid * EDGES_PER_TILE, EDGES_PER_TILE)],
        dst_v1, sems[2])
    pltpu.make_async_copy(
        edges_hbm.at[pl.ds(sid * EDGES_PER_TILE, EDGES_PER_TILE)], src_v,
        sems[1]).wait()
    pltpu.make_async_copy(
        edges_hbm.at[pl.ds(N_EDGES + sid * EDGES_PER_TILE, EDGES_PER_TILE)],
        dst_v1, sems[2]).wait()

    PER_ROW = CHUNK // LANES

    XU = 5

    def xform(k0, carry):
        for u in range(XU):
            k = k0 * XU + u
            v = src_v[pl.ds(k * LANES, LANES)]
            src_v[pl.ds(k * LANES, LANES)] = 2 * v + cid
            d = dst_v1[pl.ds(k * LANES, LANES)]
            dst_v[k // PER_ROW, pl.ds((k % PER_ROW) * LANES, LANES)] = d
        return carry

    lax.fori_loop(0, EDGES_PER_TILE // LANES // XU, xform, 0)
    pltpu.make_async_copy(
        zeros_hbm, acc.at[pl.ds(sid * ROWS_PER_TILE, ROWS_PER_TILE)],
        sems[0]).wait()
    plsc.subcore_barrier()

    for b in range(NBUF):
        pltpu.async_copy(
            x2_hbm.at[src_v.at[pl.ds(b * CHUNK, CHUNK)]], bufs[b], sems[b])

    def group(g, carry):
        for b in range(NBUF):
            j = NBUF * g + b
            pltpu.make_async_copy(
                x2_hbm.at[src_v.at[pl.ds(j * CHUNK, CHUNK)]], bufs[b],
                sems[b]).wait()
            pltpu.sync_copy(bufs[b], acc.at[dst_v.at[j]], add=True)

            @pl.when(g < NGROUP - 1)
            def _():
                pltpu.async_copy(
                    x2_hbm.at[src_v.at[pl.ds((j + NBUF) * CHUNK, CHUNK)]],
                    bufs[b], sems[b])
        return carry

    lax.fori_loop(0, NGROUP, group, 0)
    plsc.subcore_barrier()

    pltpu.sync_copy(
        acc.at[pl.ds(sid * ROWS_PER_TILE, ROWS_PER_TILE)],
        out_hbm.at[pl.ds(sid * ROWS_PER_TILE, ROWS_PER_TILE),
                   pl.ds(cid * D_HALF, D_HALF)],
    )


BLK = 2000


def _mlp_body(eps_ref, x_ref, agg_ref, w1_ref, b1_ref, w2_ref, b2_ref,
              o_ref):
    coeff = 2.0 + eps_ref[0, 0]
    h = coeff * x_ref[...] + agg_ref[...]
    a = jnp.maximum(
        jnp.dot(h, w1_ref[...], preferred_element_type=jnp.float32)
        + b1_ref[...], 0.0)
    o_ref[...] = (
        jnp.dot(a, w2_ref[...], preferred_element_type=jnp.float32)
        + b2_ref[...])


def _mlp(eps, x, agg, W1, b1, W2, b2):
    grid = (N_NODES // BLK,)
    return pl.pallas_call(
        _mlp_body,
        grid=grid,
        in_specs=[
            pl.BlockSpec((1, 1), lambda i: (0, 0)),
            pl.BlockSpec((BLK, D_FEAT), lambda i: (i, 0)),
            pl.BlockSpec((BLK, D_FEAT), lambda i: (i, 0)),
            pl.BlockSpec((D_FEAT, HIDDEN), lambda i: (0, 0)),
            pl.BlockSpec((1, HIDDEN), lambda i: (0, 0)),
            pl.BlockSpec((HIDDEN, HIDDEN), lambda i: (0, 0)),
            pl.BlockSpec((1, HIDDEN), lambda i: (0, 0)),
        ],
        out_specs=pl.BlockSpec((BLK, HIDDEN), lambda i: (i, 0)),
        out_shape=jax.ShapeDtypeStruct((N_NODES, HIDDEN), jnp.float32),
    )(eps, x, agg, W1, b1, W2, b2)


def kernel(x, edge_index, edge_attr, eps, W1, b1, W2, b2):
    edges = edge_index.astype(jnp.int32).reshape(2 * N_EDGES)
    x2 = x.reshape(2 * N_NODES, D_HALF)
    zeros = jnp.zeros((ROWS_PER_TILE, D_HALF), dtype=jnp.float32)

    agg = _sc_segment_sum(x2, edges, zeros)

    eps2 = eps.reshape(1, 1)
    b1r = b1.reshape(1, HIDDEN)
    b2r = b2.reshape(1, HIDDEN)
    return _mlp(eps2, x, agg, W1, b1r, W2, b2r)
